# baseline (speedup 1.0000x reference)
import functools

import jax
import jax.numpy as jnp
from jax import lax
from jax.experimental import pallas as pl
from jax.experimental.pallas import tpu as pltpu
from jax.experimental.pallas import tpu_sc as plsc

T = 2048
D = 1024
F = 3584
E = 8
HP = 5099
CAP = 256
FC = 512
NF = F // FC
NC = 2
NS = 16
NW = NC * NS
RPW = T // NW
DUMMY = E * CAP
CH = 16
NCH = T // CH


SLICE = T // NS
NCH_T = SLICE // CH
SPILL = T + 128
HALF = T // 2
RPH = HALF // NW


def _build_route_gather(row_base, with_gidx):
    mesh = plsc.VectorSubcoreMesh(core_axis_name="c", subcore_axis_name="s")
    if with_gidx:
        out_type = [jax.ShapeDtypeStruct((HALF, D), jnp.float32),
                    jax.ShapeDtypeStruct((T,), jnp.int32)]
    else:
        out_type = jax.ShapeDtypeStruct((HALF, D), jnp.float32)

    @functools.partial(
        pl.kernel,
        mesh=mesh,
        out_type=out_type,
        compiler_params=pltpu.CompilerParams(needs_layout_passes=False),
        scratch_types=[
            pltpu.VMEM((SLICE,), jnp.int32),
            pltpu.VMEM((CH,), jnp.int32),
            pltpu.VMEM((CH,), jnp.int32),
            pltpu.VMEM((NS * CH,), jnp.int32),
            pltpu.VMEM((SLICE,), jnp.int32),
            pltpu.VMEM((SLICE,), jnp.int32),
            pltpu.VMEM((SLICE,), jnp.int32),
            pltpu.VMEM((SLICE,), jnp.int32),
            pltpu.VMEM((RPH,), jnp.int32),
            pltpu.VMEM((RPH, D), jnp.float32),
            pltpu.SMEM((E,), jnp.int32),
            pltpu.VMEM_SHARED((NS * CH,), jnp.int32),
            pltpu.VMEM_SHARED((SPILL,), jnp.int32),
            pltpu.SemaphoreType.DMA,
        ],
    )
    def k1(tok_hbm, x_hbm, xg_hbm, *rest):
        if with_gidx:
            (gidx_hbm, tok_v, cnt_v, base_v, all_v, sidx_v, vals_v, gidx_v,
             zero_v, idx_v, rows_v, cnt_s, sp_cnt, sp_src, sem) = rest
        else:
            (tok_v, cnt_v, base_v, all_v, sidx_v, vals_v, gidx_v,
             zero_v, idx_v, rows_v, cnt_s, sp_cnt, sp_src, sem) = rest
        cid = lax.axis_index("c")
        sid = lax.axis_index("s")
        tbase = sid * SLICE
        pltpu.sync_copy(tok_hbm.at[pl.ds(tbase, SLICE)], tok_v)
        lanes = lax.iota(jnp.int32, CH)
        for e in range(E):
            cnt_s[e] = 0
        for j in range(NCH_T):
            zero_v[pl.ds(j * CH, CH)] = jnp.zeros((CH,), jnp.int32)

        def count_body(i, carry):
            t = tok_v[pl.ds(i * CH, CH)]
            eid = lax.rem(lax.rem(t, HP), E)
            for e in range(E):
                cnt_s[e] = cnt_s[e] + jnp.sum((eid == e).astype(jnp.int32))
            return carry

        lax.fori_loop(0, NCH_T, count_body, 0)
        cv = jnp.zeros((CH,), jnp.int32)
        for e in range(E):
            cv = jnp.where(lanes == e, cnt_s[e], cv)
        cnt_v[...] = cv
        pltpu.sync_copy(cnt_v, sp_cnt.at[pl.ds(sid * CH, CH)])
        pltpu.sync_copy(zero_v, sp_src.at[pl.ds(sid * SLICE, SLICE)])
        plsc.subcore_barrier()

        pltpu.sync_copy(sp_cnt, all_v)
        base = jnp.zeros((CH,), jnp.int32)
        for s2 in range(NS):
            row = all_v[pl.ds(s2 * CH, CH)]
            base = base + row * jnp.where(s2 < sid, 1, 0)
        base_v[...] = base
        bvec = base_v[...]
        for e in range(E):
            cnt_s[e] = bvec[e]

        def route_body(i, carry):
            t = tok_v[pl.ds(i * CH, CH)]
            eid = lax.rem(lax.rem(t, HP), E)
            pos = jnp.zeros((CH,), jnp.int32)
            for e in range(E):
                m = eid == e
                mi = m.astype(jnp.int32)
                pf = plsc.cumsum(mi)
                c = cnt_s[e]
                pos = jnp.where(m, pf - 1 + c, pos)
                cnt_s[e] = c + jnp.sum(mi)
            keep = pos < CAP
            slot = eid * CAP + pos
            gidx_v[pl.ds(i * CH, CH)] = jnp.where(keep, slot, DUMMY)
            sidx_v[pl.ds(i * CH, CH)] = jnp.where(keep, slot, T)
            vals_v[pl.ds(i * CH, CH)] = tbase + i * CH + lanes
            return carry

        lax.fori_loop(0, NCH_T, route_body, 0)
        pltpu.sync_copy(vals_v, sp_src.at[sidx_v])
        plsc.subcore_barrier()

        gbase = cid * (HALF // NC) + sid * RPH
        pltpu.sync_copy(sp_src.at[pl.ds(row_base + gbase, RPH)], idx_v)
        pltpu.async_copy(x_hbm.at[idx_v], rows_v, sem).wait()
        pltpu.sync_copy(rows_v, xg_hbm.at[pl.ds(gbase, RPH)])

        if with_gidx:
            @pl.when(cid == 0)
            def _():
                pltpu.sync_copy(gidx_v, gidx_hbm.at[pl.ds(tbase, SLICE)])

    return k1


NEH = E // 2


def _ffn_compute(x_ref, wk_ref, wv_ref, wr_ref, out_ref, acc_ref, f):
    xe = x_ref[...].astype(jnp.bfloat16)
    hpre = lax.dot_general(xe, wk_ref[0].astype(jnp.bfloat16),
                           (((1,), (1,)), ((), ())),
                           preferred_element_type=jnp.float32)
    h = jnp.maximum(hpre, 0.0) ** 2
    pk = lax.dot_general(h.astype(jnp.bfloat16),
                         wv_ref[0].astype(jnp.bfloat16),
                         (((1,), (1,)), ((), ())),
                         preferred_element_type=jnp.float32)

    @pl.when(f == 0)
    def _():
        acc_ref[...] = pk

    @pl.when(f > 0)
    def _():
        acc_ref[...] += pk

    @pl.when(f == NF - 1)
    def _():
        r = jax.nn.sigmoid(
            lax.dot_general(xe, wr_ref[0].astype(jnp.bfloat16),
                            (((1,), (1,)), ((), ())),
                            preferred_element_type=jnp.float32))
        out_ref[...] = r * acc_ref[...]


def _build_ffn_a():
    def body(x_ref, wk_ref, wv_ref, wr_ref, out_ref, acc_ref):
        _ffn_compute(x_ref, wk_ref, wv_ref, wr_ref, out_ref, acc_ref,
                     pl.program_id(1))

    return pl.pallas_call(
        body,
        grid=(NEH, NF),
        in_specs=[
            pl.BlockSpec((CAP, D), lambda e, f: (e, 0)),
            pl.BlockSpec((1, FC, D), lambda e, f: (e, f, 0)),
            pl.BlockSpec((1, D, FC), lambda e, f: (e, 0, f)),
            pl.BlockSpec((1, D, D), lambda e, f: (e, 0, 0)),
        ],
        out_specs=pl.BlockSpec((CAP, D), lambda e, f: (e, 0)),
        out_shape=jax.ShapeDtypeStruct((NEH * CAP, D), jnp.float32),
        scratch_shapes=[pltpu.VMEM((CAP, D), jnp.float32)],
    )


def _build_ffn_b():
    def body(x_ref, wk_ref, wv_ref, wr_ref, ylo_ref, out_ref, acc_ref):
        e = pl.program_id(0)
        f = pl.program_id(1)

        @pl.when(jnp.logical_and(e < NEH, f == 0))
        def _():
            out_ref[...] = ylo_ref[...]

        @pl.when(jnp.logical_and(e == E, f == 0))
        def _():
            out_ref[...] = jnp.zeros_like(out_ref)

        @pl.when(jnp.logical_and(e >= NEH, e < E))
        def _():
            _ffn_compute(x_ref, wk_ref, wv_ref, wr_ref, out_ref, acc_ref, f)

    def _ec(e):
        return jnp.clip(e, NEH, E - 1)

    def _ff(e, f):
        return jnp.where(e < NEH, 0, jnp.where(e >= E, NF - 1, f))

    return pl.pallas_call(
        body,
        grid=(E + 1, NF),
        in_specs=[
            pl.BlockSpec((CAP, D), lambda e, f: (jnp.clip(e - NEH, 0, NEH - 1), 0)),
            pl.BlockSpec((1, FC, D), lambda e, f: (_ec(e), _ff(e, f), 0)),
            pl.BlockSpec((1, D, FC), lambda e, f: (_ec(e), 0, _ff(e, f))),
            pl.BlockSpec((1, D, D), lambda e, f: (_ec(e), 0, 0)),
            pl.BlockSpec((CAP, D), lambda e, f: (jnp.minimum(e, NEH - 1), 0)),
        ],
        out_specs=pl.BlockSpec((CAP, D), lambda e, f: (e, 0)),
        out_shape=jax.ShapeDtypeStruct(((E + 1) * CAP, D), jnp.float32),
        scratch_shapes=[pltpu.VMEM((CAP, D), jnp.float32)],
    )


def _build_ungather():
    mesh = plsc.VectorSubcoreMesh(core_axis_name="c", subcore_axis_name="s")

    @functools.partial(
        pl.kernel,
        mesh=mesh,
        out_type=jax.ShapeDtypeStruct((T, D), jnp.float32),
        compiler_params=pltpu.CompilerParams(needs_layout_passes=False),
        scratch_types=[
            pltpu.VMEM((RPW,), jnp.int32),
            pltpu.VMEM((RPW, D), jnp.float32),
            pltpu.SemaphoreType.DMA,
        ],
    )
    def k3(yg_hbm, gidx_hbm, out_hbm, idx_v, rows_v, sem):
        wid = lax.axis_index("s") * NC + lax.axis_index("c")
        base = wid * RPW
        pltpu.sync_copy(gidx_hbm.at[pl.ds(base, RPW)], idx_v)
        pltpu.async_copy(yg_hbm.at[idx_v], rows_v, sem).wait()
        pltpu.sync_copy(rows_v, out_hbm.at[pl.ds(base, RPW)])

    return k3


_route_gather_a = _build_route_gather(0, True)
_route_gather_b = _build_route_gather(HALF, False)
_ffn_a = _build_ffn_a()
_ffn_b = _build_ffn_b()
_ungather = _build_ungather()


def kernel(x, token_ids, last_state, Wk, Wr, Wv):
    xg_lo, gidx = _route_gather_a(token_ids, x)
    xg_hi = _route_gather_b(token_ids, x)
    yg_lo = _ffn_a(xg_lo, Wk, Wv, Wr)
    yg = _ffn_b(xg_hi, Wk, Wv, Wr, yg_lo)
    return _ungather(yg, gidx)


# speedup vs baseline: 1.1680x; 1.1680x over previous
import functools

import jax
import jax.numpy as jnp
from jax import lax
from jax.experimental import pallas as pl
from jax.experimental.pallas import tpu as pltpu
from jax.experimental.pallas import tpu_sc as plsc

T = 2048
D = 1024
F = 3584
E = 8
HP = 5099
CAP = 256
FC = 896
NF = F // FC
NC = 2
NS = 16
NW = NC * NS
RPW = T // NW
DUMMY = E * CAP
CH = 16
NCH = T // CH


SLICE = T // NS
NCH_T = SLICE // CH
SPILL = T + 128


def _build_route_gather():
    mesh = plsc.VectorSubcoreMesh(core_axis_name="c", subcore_axis_name="s")

    @functools.partial(
        pl.kernel,
        mesh=mesh,
        out_type=[
            jax.ShapeDtypeStruct((T, D), jnp.float32),
            jax.ShapeDtypeStruct((T,), jnp.int32),
        ],
        compiler_params=pltpu.CompilerParams(needs_layout_passes=False),
        scratch_types=[
            pltpu.VMEM((SLICE,), jnp.int32),
            pltpu.VMEM((CH,), jnp.int32),
            pltpu.VMEM((CH,), jnp.int32),
            pltpu.VMEM((NS * CH,), jnp.int32),
            pltpu.VMEM((SLICE,), jnp.int32),
            pltpu.VMEM((SLICE,), jnp.int32),
            pltpu.VMEM((SLICE,), jnp.int32),
            pltpu.VMEM((SLICE,), jnp.int32),
            pltpu.VMEM((RPW,), jnp.int32),
            pltpu.VMEM((RPW, D), jnp.float32),
            pltpu.SMEM((E,), jnp.int32),
            pltpu.VMEM_SHARED((NS * CH,), jnp.int32),
            pltpu.VMEM_SHARED((SPILL,), jnp.int32),
            pltpu.SemaphoreType.DMA,
        ],
    )
    def k1(tok_hbm, x_hbm, xg_hbm, gidx_hbm, tok_v, cnt_v, base_v, all_v,
           sidx_v, vals_v, gidx_v, zero_v, idx_v, rows_v, cnt_s, sp_cnt,
           sp_src, sem):
        cid = lax.axis_index("c")
        sid = lax.axis_index("s")
        tbase = sid * SLICE
        pltpu.sync_copy(tok_hbm.at[pl.ds(tbase, SLICE)], tok_v)
        lanes = lax.iota(jnp.int32, CH)
        for e in range(E):
            cnt_s[e] = 0
        for j in range(NCH_T):
            zero_v[pl.ds(j * CH, CH)] = jnp.zeros((CH,), jnp.int32)

        def count_body(i, carry):
            t = tok_v[pl.ds(i * CH, CH)]
            eid = lax.rem(lax.rem(t, HP), E)
            for e in range(E):
                cnt_s[e] = cnt_s[e] + jnp.sum((eid == e).astype(jnp.int32))
            return carry

        lax.fori_loop(0, NCH_T, count_body, 0)
        cv = jnp.zeros((CH,), jnp.int32)
        for e in range(E):
            cv = jnp.where(lanes == e, cnt_s[e], cv)
        cnt_v[...] = cv
        pltpu.sync_copy(cnt_v, sp_cnt.at[pl.ds(sid * CH, CH)])
        pltpu.sync_copy(zero_v, sp_src.at[pl.ds(sid * SLICE, SLICE)])
        plsc.subcore_barrier()

        pltpu.sync_copy(sp_cnt, all_v)
        base = jnp.zeros((CH,), jnp.int32)
        for s2 in range(NS):
            row = all_v[pl.ds(s2 * CH, CH)]
            base = base + row * jnp.where(s2 < sid, 1, 0)
        base_v[...] = base
        bvec = base_v[...]
        for e in range(E):
            cnt_s[e] = bvec[e]

        def route_body(i, carry):
            t = tok_v[pl.ds(i * CH, CH)]
            eid = lax.rem(lax.rem(t, HP), E)
            pos = jnp.zeros((CH,), jnp.int32)
            for e in range(E):
                m = eid == e
                mi = m.astype(jnp.int32)
                pf = plsc.cumsum(mi)
                c = cnt_s[e]
                pos = jnp.where(m, pf - 1 + c, pos)
                cnt_s[e] = c + jnp.sum(mi)
            keep = pos < CAP
            slot = eid * CAP + pos
            gidx_v[pl.ds(i * CH, CH)] = jnp.where(keep, slot, DUMMY)
            sidx_v[pl.ds(i * CH, CH)] = jnp.where(keep, slot, T)
            vals_v[pl.ds(i * CH, CH)] = tbase + i * CH + lanes
            return carry

        lax.fori_loop(0, NCH_T, route_body, 0)
        pltpu.sync_copy(vals_v, sp_src.at[sidx_v])
        plsc.subcore_barrier()

        gbase = cid * (T // NC) + sid * RPW
        pltpu.sync_copy(sp_src.at[pl.ds(gbase, RPW)], idx_v)
        pltpu.async_copy(x_hbm.at[idx_v], rows_v, sem).wait()
        pltpu.sync_copy(rows_v, xg_hbm.at[pl.ds(gbase, RPW)])

        @pl.when(cid == 0)
        def _():
            pltpu.sync_copy(gidx_v, gidx_hbm.at[pl.ds(tbase, SLICE)])

    return k1


def _build_ffn():
    def body(x_ref, wk_ref, wv_ref, wr_ref, out_ref, acc_ref):
        e = pl.program_id(0)
        f = pl.program_id(1)

        @pl.when(jnp.logical_and(e == E, f == 0))
        def _():
            out_ref[...] = jnp.zeros_like(out_ref)

        @pl.when(e < E)
        def _():
            xe = x_ref[...].astype(jnp.bfloat16)
            hpre = lax.dot_general(xe, wk_ref[0].astype(jnp.bfloat16),
                                   (((1,), (1,)), ((), ())),
                                   preferred_element_type=jnp.float32)
            h = jnp.maximum(hpre, 0.0) ** 2
            pk = lax.dot_general(h.astype(jnp.bfloat16),
                                 wv_ref[0].astype(jnp.bfloat16),
                                 (((1,), (1,)), ((), ())),
                                 preferred_element_type=jnp.float32)

            @pl.when(f == 0)
            def _():
                acc_ref[...] = pk

            @pl.when(f > 0)
            def _():
                acc_ref[...] += pk

            @pl.when(f == NF - 1)
            def _():
                r = jax.nn.sigmoid(
                    lax.dot_general(xe, wr_ref[0].astype(jnp.bfloat16),
                                    (((1,), (1,)), ((), ())),
                                    preferred_element_type=jnp.float32))
                out_ref[...] = r * acc_ref[...]

    def _ec(e):
        return jnp.minimum(e, E - 1)

    def _fc(e, f):
        return jnp.where(e >= E, NF - 1, f)

    return pl.pallas_call(
        body,
        grid=(E + 1, NF),
        in_specs=[
            pl.BlockSpec((CAP, D), lambda e, f: (_ec(e), 0)),
            pl.BlockSpec((1, FC, D), lambda e, f: (_ec(e), _fc(e, f), 0)),
            pl.BlockSpec((1, D, FC), lambda e, f: (_ec(e), 0, _fc(e, f))),
            pl.BlockSpec((1, D, D), lambda e, f: (_ec(e), 0, 0)),
        ],
        out_specs=pl.BlockSpec((CAP, D), lambda e, f: (e, 0)),
        out_shape=jax.ShapeDtypeStruct(((E + 1) * CAP, D), jnp.float32),
        scratch_shapes=[pltpu.VMEM((CAP, D), jnp.float32)],
    )


def _build_ungather():
    mesh = plsc.VectorSubcoreMesh(core_axis_name="c", subcore_axis_name="s")

    @functools.partial(
        pl.kernel,
        mesh=mesh,
        out_type=jax.ShapeDtypeStruct((T, D), jnp.float32),
        compiler_params=pltpu.CompilerParams(needs_layout_passes=False),
        scratch_types=[
            pltpu.VMEM((RPW,), jnp.int32),
            pltpu.VMEM((RPW, D), jnp.float32),
            pltpu.SemaphoreType.DMA,
        ],
    )
    def k3(yg_hbm, gidx_hbm, out_hbm, idx_v, rows_v, sem):
        wid = lax.axis_index("s") * NC + lax.axis_index("c")
        base = wid * RPW
        pltpu.sync_copy(gidx_hbm.at[pl.ds(base, RPW)], idx_v)
        pltpu.async_copy(yg_hbm.at[idx_v], rows_v, sem).wait()
        pltpu.sync_copy(rows_v, out_hbm.at[pl.ds(base, RPW)])

    return k3


_route_gather = _build_route_gather()
_ffn = _build_ffn()
_ungather = _build_ungather()


def kernel(x, token_ids, last_state, Wk, Wr, Wv):
    xg, gidx = _route_gather(token_ids, x)
    yg = _ffn(xg, Wk, Wv, Wr)
    return _ungather(yg, gidx)


# speedup vs baseline: 1.2132x; 1.0387x over previous
import functools

import jax
import jax.numpy as jnp
from jax import lax
from jax.experimental import pallas as pl
from jax.experimental.pallas import tpu as pltpu
from jax.experimental.pallas import tpu_sc as plsc

T = 2048
D = 1024
F = 3584
E = 8
HP = 5099
CAP = 256
FC = 1792
NF = F // FC
NC = 2
NS = 16
NW = NC * NS
RPW = T // NW
DUMMY = E * CAP
CH = 16
NCH = T // CH


SLICE = T // NS
NCH_T = SLICE // CH
SPILL = T + 128


def _build_route_gather():
    mesh = plsc.VectorSubcoreMesh(core_axis_name="c", subcore_axis_name="s")

    @functools.partial(
        pl.kernel,
        mesh=mesh,
        out_type=[
            jax.ShapeDtypeStruct((T, D), jnp.float32),
            jax.ShapeDtypeStruct((T,), jnp.int32),
        ],
        compiler_params=pltpu.CompilerParams(needs_layout_passes=False),
        scratch_types=[
            pltpu.VMEM((SLICE,), jnp.int32),
            pltpu.VMEM((CH,), jnp.int32),
            pltpu.VMEM((CH,), jnp.int32),
            pltpu.VMEM((NS * CH,), jnp.int32),
            pltpu.VMEM((SLICE,), jnp.int32),
            pltpu.VMEM((SLICE,), jnp.int32),
            pltpu.VMEM((SLICE,), jnp.int32),
            pltpu.VMEM((SLICE,), jnp.int32),
            pltpu.VMEM((RPW,), jnp.int32),
            pltpu.VMEM((RPW, D), jnp.float32),
            pltpu.SMEM((E,), jnp.int32),
            pltpu.VMEM_SHARED((NS * CH,), jnp.int32),
            pltpu.VMEM_SHARED((SPILL,), jnp.int32),
            pltpu.SemaphoreType.DMA,
        ],
    )
    def k1(tok_hbm, x_hbm, xg_hbm, gidx_hbm, tok_v, cnt_v, base_v, all_v,
           sidx_v, vals_v, gidx_v, zero_v, idx_v, rows_v, cnt_s, sp_cnt,
           sp_src, sem):
        cid = lax.axis_index("c")
        sid = lax.axis_index("s")
        tbase = sid * SLICE
        pltpu.sync_copy(tok_hbm.at[pl.ds(tbase, SLICE)], tok_v)
        lanes = lax.iota(jnp.int32, CH)
        for e in range(E):
            cnt_s[e] = 0
        for j in range(NCH_T):
            zero_v[pl.ds(j * CH, CH)] = jnp.zeros((CH,), jnp.int32)

        def count_body(i, carry):
            t = tok_v[pl.ds(i * CH, CH)]
            eid = lax.rem(lax.rem(t, HP), E)
            for e in range(E):
                cnt_s[e] = cnt_s[e] + jnp.sum((eid == e).astype(jnp.int32))
            return carry

        lax.fori_loop(0, NCH_T, count_body, 0)
        cv = jnp.zeros((CH,), jnp.int32)
        for e in range(E):
            cv = jnp.where(lanes == e, cnt_s[e], cv)
        cnt_v[...] = cv
        pltpu.sync_copy(cnt_v, sp_cnt.at[pl.ds(sid * CH, CH)])
        pltpu.sync_copy(zero_v, sp_src.at[pl.ds(sid * SLICE, SLICE)])
        plsc.subcore_barrier()

        pltpu.sync_copy(sp_cnt, all_v)
        base = jnp.zeros((CH,), jnp.int32)
        for s2 in range(NS):
            row = all_v[pl.ds(s2 * CH, CH)]
            base = base + row * jnp.where(s2 < sid, 1, 0)
        base_v[...] = base
        bvec = base_v[...]
        for e in range(E):
            cnt_s[e] = bvec[e]

        def route_body(i, carry):
            t = tok_v[pl.ds(i * CH, CH)]
            eid = lax.rem(lax.rem(t, HP), E)
            pos = jnp.zeros((CH,), jnp.int32)
            for e in range(E):
                m = eid == e
                mi = m.astype(jnp.int32)
                pf = plsc.cumsum(mi)
                c = cnt_s[e]
                pos = jnp.where(m, pf - 1 + c, pos)
                cnt_s[e] = c + jnp.sum(mi)
            keep = pos < CAP
            slot = eid * CAP + pos
            gidx_v[pl.ds(i * CH, CH)] = jnp.where(keep, slot, DUMMY)
            sidx_v[pl.ds(i * CH, CH)] = jnp.where(keep, slot, T)
            vals_v[pl.ds(i * CH, CH)] = tbase + i * CH + lanes
            return carry

        lax.fori_loop(0, NCH_T, route_body, 0)
        pltpu.sync_copy(vals_v, sp_src.at[sidx_v])
        plsc.subcore_barrier()

        gbase = cid * (T // NC) + sid * RPW
        pltpu.sync_copy(sp_src.at[pl.ds(gbase, RPW)], idx_v)
        pltpu.async_copy(x_hbm.at[idx_v], rows_v, sem).wait()
        pltpu.sync_copy(rows_v, xg_hbm.at[pl.ds(gbase, RPW)])

        @pl.when(cid == 0)
        def _():
            pltpu.sync_copy(gidx_v, gidx_hbm.at[pl.ds(tbase, SLICE)])

    return k1


def _build_ffn():
    def body(x_ref, wk_ref, wv_ref, wr_ref, out_ref, acc_ref):
        e = pl.program_id(0)
        f = pl.program_id(1)

        @pl.when(jnp.logical_and(e == E, f == 0))
        def _():
            out_ref[...] = jnp.zeros_like(out_ref)

        @pl.when(e < E)
        def _():
            xe = x_ref[...].astype(jnp.bfloat16)
            hpre = lax.dot_general(xe, wk_ref[0].astype(jnp.bfloat16),
                                   (((1,), (1,)), ((), ())),
                                   preferred_element_type=jnp.float32)
            h = jnp.maximum(hpre, 0.0) ** 2
            pk = lax.dot_general(h.astype(jnp.bfloat16),
                                 wv_ref[0].astype(jnp.bfloat16),
                                 (((1,), (1,)), ((), ())),
                                 preferred_element_type=jnp.float32)

            @pl.when(f == 0)
            def _():
                acc_ref[...] = pk

            @pl.when(f > 0)
            def _():
                acc_ref[...] += pk

            @pl.when(f == NF - 1)
            def _():
                r = jax.nn.sigmoid(
                    lax.dot_general(xe, wr_ref[0].astype(jnp.bfloat16),
                                    (((1,), (1,)), ((), ())),
                                    preferred_element_type=jnp.float32))
                out_ref[...] = r * acc_ref[...]

    def _ec(e):
        return jnp.minimum(e, E - 1)

    def _fc(e, f):
        return jnp.where(e >= E, NF - 1, f)

    return pl.pallas_call(
        body,
        grid=(E + 1, NF),
        in_specs=[
            pl.BlockSpec((CAP, D), lambda e, f: (_ec(e), 0)),
            pl.BlockSpec((1, FC, D), lambda e, f: (_ec(e), _fc(e, f), 0)),
            pl.BlockSpec((1, D, FC), lambda e, f: (_ec(e), 0, _fc(e, f))),
            pl.BlockSpec((1, D, D), lambda e, f: (_ec(e), 0, 0)),
        ],
        out_specs=pl.BlockSpec((CAP, D), lambda e, f: (e, 0)),
        out_shape=jax.ShapeDtypeStruct(((E + 1) * CAP, D), jnp.float32),
        scratch_shapes=[pltpu.VMEM((CAP, D), jnp.float32)],
    )


def _build_ungather():
    mesh = plsc.VectorSubcoreMesh(core_axis_name="c", subcore_axis_name="s")

    @functools.partial(
        pl.kernel,
        mesh=mesh,
        out_type=jax.ShapeDtypeStruct((T, D), jnp.float32),
        compiler_params=pltpu.CompilerParams(needs_layout_passes=False),
        scratch_types=[
            pltpu.VMEM((RPW,), jnp.int32),
            pltpu.VMEM((RPW, D), jnp.float32),
            pltpu.SemaphoreType.DMA,
        ],
    )
    def k3(yg_hbm, gidx_hbm, out_hbm, idx_v, rows_v, sem):
        wid = lax.axis_index("s") * NC + lax.axis_index("c")
        base = wid * RPW
        pltpu.sync_copy(gidx_hbm.at[pl.ds(base, RPW)], idx_v)
        pltpu.async_copy(yg_hbm.at[idx_v], rows_v, sem).wait()
        pltpu.sync_copy(rows_v, out_hbm.at[pl.ds(base, RPW)])

    return k3


_route_gather = _build_route_gather()
_ffn = _build_ffn()
_ungather = _build_ungather()


def kernel(x, token_ids, last_state, Wk, Wr, Wv):
    xg, gidx = _route_gather(token_ids, x)
    yg = _ffn(xg, Wk, Wv, Wr)
    return _ungather(yg, gidx)
